# baseline (device time: 36293 ns/iter reference)
import jax
import jax.numpy as jnp
from jax import lax
from jax.experimental import pallas as pl
from jax.experimental.pallas import tpu as pltpu

N_DEV = 16
N_TOK = 512
D_OUT = 512
E_PER = 2
CHUNK = N_TOK // N_DEV
N_STEP = 4
N_FLOW = 2
COL_H = D_OUT // N_FLOW

ORDERS = ((0, 2, 1, 3), (2, 0, 3, 1))

RS_ROWS = [(1 << (N_STEP - 1 - t)) * CHUNK for t in range(N_STEP)]
RS_OFF = [sum(RS_ROWS[:t]) for t in range(N_STEP)]
BUF_ROWS = sum(RS_ROWS)


def _virt(d, order):
    v = 0
    for t, k in enumerate(order):
        v = v + ((d >> k) & 1) * (1 << (N_STEP - 1 - t))
    return v


_PERMS = [[_virt(c, o) for c in range(N_DEV)] for o in ORDERS]


def kernel(x, router_W, route_idx, expert_W):
    del router_W

    def body(x_ref, idx_ref, w_ref, out_ref, acc_ref, bf_buf,
             rs_send, rs_recv, ag_send, ag_recv):
        my = lax.axis_index("i")
        virt = [_virt(my, o) for o in ORDERS]

        barrier = pltpu.get_barrier_semaphore()
        for b in range(N_STEP):
            pl.semaphore_signal(barrier, inc=1, device_id=(my ^ (1 << b),),
                                device_id_type=pl.DeviceIdType.MESH)
        pl.semaphore_wait(barrier, N_STEP)

        route = idx_ref[:, :]
        x_all = x_ref[:, :]
        e0 = my * E_PER
        partial = jnp.zeros((N_TOK, D_OUT), jnp.float32)
        for k in range(E_PER):
            mask = (route == (e0 + k)).astype(jnp.float32)
            partial = partial + jnp.dot(
                x_all * mask, w_ref[k], preferred_element_type=jnp.float32)

        for f in range(N_FLOW):
            for c in range(N_DEV):
                acc_ref[f, pl.ds(_PERMS[f][c] * CHUNK, CHUNK), :] = lax.slice(
                    partial, (c * CHUNK, f * COL_H),
                    ((c + 1) * CHUNK, (f + 1) * COL_H))

        for t in range(N_STEP):
            vb = N_STEP - 1 - t
            rows = RS_ROWS[t]
            rdmas = []
            for f in range(N_FLOW):
                send = (((virt[f] >> vb) << vb) ^ (1 << vb)) * CHUNK
                rdma = pltpu.make_async_remote_copy(
                    src_ref=acc_ref.at[f, pl.ds(send, rows), :],
                    dst_ref=bf_buf.at[f, pl.ds(RS_OFF[t], rows), :],
                    send_sem=rs_send.at[t, f],
                    recv_sem=rs_recv.at[t, f],
                    device_id=(my ^ (1 << ORDERS[f][t]),),
                    device_id_type=pl.DeviceIdType.MESH,
                )
                rdma.start()
                rdmas.append(rdma)
            for f in range(N_FLOW):
                rdmas[f].wait()
                keep = ((virt[f] >> vb) << vb) * CHUNK
                acc_ref[f, pl.ds(keep, rows), :] = (
                    acc_ref[f, pl.ds(keep, rows), :]
                    + bf_buf[f, pl.ds(RS_OFF[t], rows), :])

        for u in range(N_STEP):
            rows = (1 << u) * CHUNK
            rdmas = []
            for f in range(N_FLOW):
                send = ((virt[f] >> u) << u) * CHUNK
                rdma = pltpu.make_async_remote_copy(
                    src_ref=acc_ref.at[f, pl.ds(send, rows), :],
                    dst_ref=acc_ref.at[f, pl.ds(send, rows), :],
                    send_sem=ag_send.at[u, f],
                    recv_sem=ag_recv.at[u, f],
                    device_id=(my ^ (1 << ORDERS[f][N_STEP - 1 - u]),),
                    device_id_type=pl.DeviceIdType.MESH,
                )
                rdma.start()
                rdmas.append(rdma)
            for rdma in rdmas:
                rdma.wait()

        for f in range(N_FLOW):
            for c in range(N_DEV):
                out_ref[pl.ds(c * CHUNK, CHUNK),
                        pl.ds(f * COL_H, COL_H)] = acc_ref[
                    f, pl.ds(_PERMS[f][c] * CHUNK, CHUNK), :]

    return pl.pallas_call(
        body,
        out_shape=jax.ShapeDtypeStruct((N_TOK, D_OUT), jnp.float32),
        in_specs=[
            pl.BlockSpec(memory_space=pltpu.VMEM),
            pl.BlockSpec(memory_space=pltpu.VMEM),
            pl.BlockSpec(memory_space=pltpu.VMEM),
        ],
        out_specs=pl.BlockSpec(memory_space=pltpu.VMEM),
        scratch_shapes=[
            pltpu.VMEM((N_FLOW, N_TOK, COL_H), jnp.float32),
            pltpu.VMEM((N_FLOW, BUF_ROWS, COL_H), jnp.float32),
            pltpu.SemaphoreType.DMA((N_STEP, N_FLOW)),
            pltpu.SemaphoreType.DMA((N_STEP, N_FLOW)),
            pltpu.SemaphoreType.DMA((N_STEP, N_FLOW)),
            pltpu.SemaphoreType.DMA((N_STEP, N_FLOW)),
        ],
        compiler_params=pltpu.CompilerParams(collective_id=0),
    )(x, route_idx, expert_W)


# device time: 30639 ns/iter; 1.1845x vs baseline; 1.1845x over previous
import jax
import jax.numpy as jnp
from jax import lax
from jax.experimental import pallas as pl
from jax.experimental.pallas import tpu as pltpu

N_DEV = 16
N_TOK = 512
D_OUT = 512
E_PER = 2
CHUNK = N_TOK // N_DEV
N_STEP = 4
N_FLOW = 2
COL_H = D_OUT // N_FLOW

ORDERS = ((0, 2, 1, 3), (2, 0, 3, 1))

RS_ROWS = [(1 << (N_STEP - 1 - t)) * CHUNK for t in range(N_STEP)]
RS_OFF = [sum(RS_ROWS[:t]) for t in range(N_STEP)]
BUF_ROWS = sum(RS_ROWS)


def _virt(d, order):
    v = 0
    for t, k in enumerate(order):
        v = v + ((d >> k) & 1) * (1 << (N_STEP - 1 - t))
    return v


_PERMS = [[_virt(c, o) for c in range(N_DEV)] for o in ORDERS]


def kernel(x, router_W, route_idx, expert_W):
    del router_W

    def body(x_ref, idx_ref, w_ref, out_ref, acc_ref, bf_buf,
             rs_send, rs_recv, ag_send, ag_recv):
        my = lax.axis_index("i")
        virt = [_virt(my, o) for o in ORDERS]

        barrier = pltpu.get_barrier_semaphore()
        for b in range(N_STEP):
            pl.semaphore_signal(barrier, inc=1, device_id=(my ^ (1 << b),),
                                device_id_type=pl.DeviceIdType.MESH)
        pl.semaphore_wait(barrier, N_STEP)

        route = idx_ref[:, :]
        x_all = x_ref[:, :]
        e0 = my * E_PER
        partial = jnp.zeros((N_TOK, D_OUT), jnp.float32)
        for k in range(E_PER):
            mask = (route == (e0 + k)).astype(jnp.float32)
            partial = partial + jnp.dot(
                x_all * mask, w_ref[k], preferred_element_type=jnp.float32)

        partial = partial.astype(jnp.bfloat16)
        for f in range(N_FLOW):
            for c in range(N_DEV):
                acc_ref[f, pl.ds(_PERMS[f][c] * CHUNK, CHUNK), :] = lax.slice(
                    partial, (c * CHUNK, f * COL_H),
                    ((c + 1) * CHUNK, (f + 1) * COL_H))

        for t in range(N_STEP):
            vb = N_STEP - 1 - t
            rows = RS_ROWS[t]
            rdmas = []
            for f in range(N_FLOW):
                send = (((virt[f] >> vb) << vb) ^ (1 << vb)) * CHUNK
                rdma = pltpu.make_async_remote_copy(
                    src_ref=acc_ref.at[f, pl.ds(send, rows), :],
                    dst_ref=bf_buf.at[f, pl.ds(RS_OFF[t], rows), :],
                    send_sem=rs_send.at[t, f],
                    recv_sem=rs_recv.at[t, f],
                    device_id=(my ^ (1 << ORDERS[f][t]),),
                    device_id_type=pl.DeviceIdType.MESH,
                )
                rdma.start()
                rdmas.append(rdma)
            for f in range(N_FLOW):
                rdmas[f].wait()
                keep = ((virt[f] >> vb) << vb) * CHUNK
                acc_ref[f, pl.ds(keep, rows), :] = (
                    acc_ref[f, pl.ds(keep, rows), :].astype(jnp.float32)
                    + bf_buf[f, pl.ds(RS_OFF[t], rows), :].astype(jnp.float32)
                ).astype(jnp.bfloat16)

        for u in range(N_STEP):
            rows = (1 << u) * CHUNK
            rdmas = []
            for f in range(N_FLOW):
                send = ((virt[f] >> u) << u) * CHUNK
                rdma = pltpu.make_async_remote_copy(
                    src_ref=acc_ref.at[f, pl.ds(send, rows), :],
                    dst_ref=acc_ref.at[f, pl.ds(send, rows), :],
                    send_sem=ag_send.at[u, f],
                    recv_sem=ag_recv.at[u, f],
                    device_id=(my ^ (1 << ORDERS[f][N_STEP - 1 - u]),),
                    device_id_type=pl.DeviceIdType.MESH,
                )
                rdma.start()
                rdmas.append(rdma)
            for rdma in rdmas:
                rdma.wait()

        for f in range(N_FLOW):
            for c in range(N_DEV):
                out_ref[pl.ds(c * CHUNK, CHUNK),
                        pl.ds(f * COL_H, COL_H)] = acc_ref[
                    f, pl.ds(_PERMS[f][c] * CHUNK, CHUNK), :].astype(
                        jnp.float32)

    return pl.pallas_call(
        body,
        out_shape=jax.ShapeDtypeStruct((N_TOK, D_OUT), jnp.float32),
        in_specs=[
            pl.BlockSpec(memory_space=pltpu.VMEM),
            pl.BlockSpec(memory_space=pltpu.VMEM),
            pl.BlockSpec(memory_space=pltpu.VMEM),
        ],
        out_specs=pl.BlockSpec(memory_space=pltpu.VMEM),
        scratch_shapes=[
            pltpu.VMEM((N_FLOW, N_TOK, COL_H), jnp.bfloat16),
            pltpu.VMEM((N_FLOW, BUF_ROWS, COL_H), jnp.bfloat16),
            pltpu.SemaphoreType.DMA((N_STEP, N_FLOW)),
            pltpu.SemaphoreType.DMA((N_STEP, N_FLOW)),
            pltpu.SemaphoreType.DMA((N_STEP, N_FLOW)),
            pltpu.SemaphoreType.DMA((N_STEP, N_FLOW)),
        ],
        compiler_params=pltpu.CompilerParams(collective_id=0),
    )(x, route_idx, expert_W)
